# baseline (speedup 1.0000x reference)
import functools

import jax
import jax.numpy as jnp
from jax import lax
from jax.experimental import pallas as pl
from jax.experimental.pallas import tpu as pltpu
from jax.experimental.pallas import tpu_sc as plsc

B, N, H = 8, 128, 128
L = 16
NW = 32
QJ = N // 4


def _sc_er_agg_body(er_hbm, adj_hbm, red_hbm, er_vm, adj_vm, out_vm):
    wid = lax.axis_index("s") * 2 + lax.axis_index("c")
    b = wid // 4
    q = wid % 4
    pltpu.sync_copy(adj_hbm.at[pl.ds(q * QJ, QJ), :], adj_vm)
    pltpu.sync_copy(er_hbm.at[b, :, pl.ds(q * QJ * 3, QJ * 3)], er_vm)

    for chunk in range(QJ // L):
        jloc = chunk * L + lax.iota(jnp.int32, L)
        jc0 = jloc * 3
        zero = jnp.zeros((L,), jnp.float32)

        def body(i, accs):
            a0, a1, a2 = accs
            iv = jnp.full((L,), i, jnp.int32)
            m = plsc.load_gather(adj_vm, [jloc, iv])
            sel = m > 0
            e0 = plsc.load_gather(er_vm, [iv, jc0])
            e1 = plsc.load_gather(er_vm, [iv, jc0 + 1])
            e2 = plsc.load_gather(er_vm, [iv, jc0 + 2])
            a0 = a0 + jnp.where(sel, e0, 0.0)
            a1 = a1 + jnp.where(sel, e1, 0.0)
            a2 = a2 + jnp.where(sel, e2, 0.0)
            return (a0, a1, a2)

        a0, a1, a2 = lax.fori_loop(0, N, body, (zero, zero, zero))
        out_vm[0, pl.ds(chunk * L, L)] = a0
        out_vm[1, pl.ds(chunk * L, L)] = a1
        out_vm[2, pl.ds(chunk * L, L)] = a2

    for c in range(3):
        pltpu.sync_copy(out_vm.at[c], red_hbm.at[b, c, pl.ds(q * QJ, QJ)])


def _sc_er_agg(er3, adj):
    mesh = plsc.VectorSubcoreMesh(core_axis_name="c", subcore_axis_name="s")
    return pl.kernel(
        _sc_er_agg_body,
        mesh=mesh,
        compiler_params=pltpu.CompilerParams(use_tc_tiling_on_sc=False,
                                             needs_layout_passes=False),
        out_type=jax.ShapeDtypeStruct((B, 3, N), jnp.float32),
        scratch_types=[
            pltpu.VMEM((N, QJ * 3), jnp.float32),
            pltpu.VMEM((QJ, N), jnp.int32),
            pltpu.VMEM((3, QJ), jnp.float32),
        ],
    )(er3, adj)


def _fused_kernel(adj_ref, ne_ref, red_ref, wnT_ref, weT_ref,
                  bmsg_ref, wu1T_ref, wu2T_ref, bup_ref, gamma_ref, beta_ref,
                  out_ref):
    f32 = jnp.float32
    mask = (adj_ref[:] > 0).astype(f32)
    deg = jnp.sum(mask, axis=1, keepdims=True)
    bias_jh = deg * bmsg_ref[:]
    weT = weT_ref[:]
    wnT = wnT_ref[:]
    wu1T = wu1T_ref[:]
    wu2T = wu2T_ref[:]
    bup = bup_ref[:]

    s = jnp.zeros((1, H), f32)
    s2 = jnp.zeros((1, H), f32)
    for b in range(B):
        ne_b = ne_ref[b]
        red_b = red_ref[b]
        term_b = lax.dot_general(red_b, weT, (((0,), (0,)), ((), ())),
                                 preferred_element_type=f32)
        proj_b = jnp.dot(ne_b, wnT, preferred_element_type=f32)
        msg_b = jnp.dot(mask, proj_b, preferred_element_type=f32) \
            + term_b + bias_jh
        up_b = jnp.dot(ne_b, wu1T, preferred_element_type=f32) \
            + jnp.dot(msg_b, wu2T, preferred_element_type=f32) + bup
        up_b = jnp.maximum(up_b, 0.0)
        out_ref[b] = up_b
        s = s + jnp.sum(up_b, axis=0, keepdims=True)
        s2 = s2 + jnp.sum(up_b * up_b, axis=0, keepdims=True)

    inv_n = 1.0 / (B * N)
    mean = s * inv_n
    var = s2 * inv_n - mean * mean
    scale = lax.rsqrt(var + 1e-5) * gamma_ref[:]
    shift = beta_ref[:] - mean * scale
    for b in range(B):
        out_ref[b] = out_ref[b] * scale + shift + ne_ref[b]


def kernel(node_embeddings, edge_relations, adjacency, W_msg, b_msg,
           W_up, b_up, bn_gamma, bn_beta):
    ne = node_embeddings.astype(jnp.float32)
    adj = adjacency.astype(jnp.int32)
    er3 = edge_relations.astype(jnp.float32).reshape(B, N, N * 3)
    red = _sc_er_agg(er3, adj)
    wnT = W_msg[:, :H].T
    weT = W_msg[:, H:].T
    wu1T = W_up[:, :H].T
    wu2T = W_up[:, H:].T
    bmsg = b_msg.reshape(1, H)
    bup = b_up.reshape(1, H)
    gamma = bn_gamma.reshape(1, H)
    beta = bn_beta.reshape(1, H)
    return pl.pallas_call(
        _fused_kernel,
        out_shape=jax.ShapeDtypeStruct((B, N, H), jnp.float32),
    )(adj, ne, red, wnT, weT, bmsg, wu1T, wu2T, bup, gamma, beta)
